# Initial kernel scaffold; baseline (speedup 1.0000x reference)
import functools

import jax
import jax.numpy as jnp
from jax import lax
from jax.experimental import pallas as pl
from jax.experimental.pallas import tpu as pltpu
from jax.experimental.pallas import tpu_sc as plsc

N = 10000
E = 320000
D = 128
NPAD = 10240
STRIP = NPAD // 16
NC, NS = 2, 16
NW = NC * NS
EPT = E // NW
B = 80
RPT = EPT // B
EROWS = E // B

_MESH = plsc.VectorSubcoreMesh(core_axis_name="c", subcore_axis_name="s")
_f32 = jnp.float32


def _wid():
    return lax.axis_index("s") * NC + lax.axis_index("c")


@functools.partial(
    pl.kernel,
    out_type=jax.ShapeDtypeStruct((NC, NPAD), _f32),
    mesh=_MESH,
    scratch_types=[
        pltpu.VMEM((RPT, B), jnp.int32),
        pltpu.VMEM((B,), _f32),
        pltpu.VMEM_SHARED((NPAD,), _f32),
    ],
)
def _sc_degree(dst2d, zeros1, out, didx, ones_v, acc):
    c = lax.axis_index("c")
    s = lax.axis_index("s")
    pltpu.sync_copy(zeros1, acc.at[pl.ds(s * STRIP, STRIP)])
    pltpu.sync_copy(dst2d.at[pl.ds(_wid() * RPT, RPT)], didx)
    for k in range(B // 16):
        ones_v[pl.ds(k * 16, 16)] = jnp.ones((16,), _f32)
    plsc.subcore_barrier()

    def body(j, carry):
        pltpu.sync_copy(ones_v, acc.at[didx.at[j]], add=True)
        return carry

    lax.fori_loop(0, RPT, body, 0)
    plsc.subcore_barrier()
    pltpu.sync_copy(acc.at[pl.ds(s * STRIP, STRIP)],
                    out.at[c, pl.ds(s * STRIP, STRIP)])


@functools.partial(
    pl.kernel,
    out_type=jax.ShapeDtypeStruct((NC, NPAD, D), _f32),
    mesh=_MESH,
    scratch_types=[
        pltpu.VMEM((RPT, B), jnp.int32),
        pltpu.VMEM((RPT, B), jnp.int32),
        pltpu.VMEM((B, D), _f32),
        pltpu.VMEM_SHARED((NPAD, D), _f32),
        pltpu.SemaphoreType.DMA,
    ],
)
def _sc_agg(y, src2d, dst2d, zeros2, out, sidx, didx, rows, acc, sem):
    c = lax.axis_index("c")
    s = lax.axis_index("s")
    w = _wid()
    pltpu.sync_copy(zeros2, acc.at[pl.ds(s * STRIP, STRIP)])
    pltpu.sync_copy(src2d.at[pl.ds(w * RPT, RPT)], sidx)
    pltpu.sync_copy(dst2d.at[pl.ds(w * RPT, RPT)], didx)
    plsc.subcore_barrier()

    def body(j, carry):
        pltpu.async_copy(y.at[sidx.at[j]], rows, sem).wait()
        pltpu.sync_copy(rows, acc.at[didx.at[j]], add=True)
        return carry

    lax.fori_loop(0, RPT, body, 0)
    plsc.subcore_barrier()
    pltpu.sync_copy(acc.at[pl.ds(s * STRIP, STRIP)],
                    out.at[c, pl.ds(s * STRIP, STRIP)])


@functools.partial(
    pl.kernel,
    out_type=(jax.ShapeDtypeStruct((E, D), _f32),
              jax.ShapeDtypeStruct((E, D), _f32)),
    mesh=_MESH,
    scratch_types=[
        pltpu.VMEM((RPT, B), jnp.int32),
        pltpu.VMEM((RPT, B), jnp.int32),
        pltpu.VMEM((B, D), _f32),
        pltpu.VMEM((B, D), _f32),
        pltpu.SemaphoreType.DMA,
        pltpu.SemaphoreType.DMA,
    ],
)
def _sc_gather2(p, q, src2d, dst2d, pg, qg, sidx, didx, bufp, bufq, semp, semq):
    w = _wid()
    pltpu.sync_copy(src2d.at[pl.ds(w * RPT, RPT)], sidx)
    pltpu.sync_copy(dst2d.at[pl.ds(w * RPT, RPT)], didx)
    base = w * EPT

    def body(j, carry):
        cp = pltpu.async_copy(p.at[sidx.at[j]], bufp, semp)
        cq = pltpu.async_copy(q.at[didx.at[j]], bufq, semq)
        cp.wait()
        cq.wait()
        pltpu.sync_copy(bufp, pg.at[pl.ds(base + j * B, B)])
        pltpu.sync_copy(bufq, qg.at[pl.ds(base + j * B, B)])
        return carry

    lax.fori_loop(0, RPT, body, 0)


_R = 640


def _prep_body(x_ref, w_ref, d0_ref, d1_ref, y_ref, dis_ref):
    dis = lax.rsqrt(d0_ref[0] + d1_ref[0] + 1.0)[:, None]
    y_ref[...] = jnp.dot(x_ref[...], w_ref[...],
                         preferred_element_type=_f32) * dis
    dis_ref[...] = dis


def _tc_prep(x_pad, W1, degp):
    return pl.pallas_call(
        _prep_body,
        grid=(NPAD // _R,),
        in_specs=[
            pl.BlockSpec((_R, D), lambda i: (i, 0)),
            pl.BlockSpec((D, D), lambda i: (0, 0)),
            pl.BlockSpec((1, _R), lambda i: (0, i)),
            pl.BlockSpec((1, _R), lambda i: (1, i)),
        ],
        out_specs=[
            pl.BlockSpec((_R, D), lambda i: (i, 0)),
            pl.BlockSpec((_R, 1), lambda i: (i, 0)),
        ],
        out_shape=[
            jax.ShapeDtypeStruct((NPAD, D), _f32),
            jax.ShapeDtypeStruct((NPAD, 1), _f32),
        ],
    )(x_pad, W1, degp, degp)


def _layer_body(a0_ref, a1_ref, y_ref, dis_ref, b_ref, w_ref, o_ref):
    h = jnp.maximum(
        dis_ref[...] * (a0_ref[0] + a1_ref[0] + y_ref[...]) + b_ref[...], 0.0)
    o_ref[...] = jnp.dot(h, w_ref[...], preferred_element_type=_f32) * dis_ref[...]


def _tc_layer(agg, y, dis, b, W):
    return pl.pallas_call(
        _layer_body,
        grid=(NPAD // _R,),
        in_specs=[
            pl.BlockSpec((1, _R, D), lambda i: (0, i, 0)),
            pl.BlockSpec((1, _R, D), lambda i: (1, i, 0)),
            pl.BlockSpec((_R, D), lambda i: (i, 0)),
            pl.BlockSpec((_R, 1), lambda i: (i, 0)),
            pl.BlockSpec((1, D), lambda i: (0, 0)),
            pl.BlockSpec((D, D), lambda i: (0, 0)),
        ],
        out_specs=pl.BlockSpec((_R, D), lambda i: (i, 0)),
        out_shape=jax.ShapeDtypeStruct((NPAD, D), _f32),
    )(agg, agg, y, dis, b, W)


def _pq_body(a0_ref, a1_ref, y_ref, dis_ref, b_ref, wa_ref, wb_ref,
             p_ref, q_ref):
    h = jnp.maximum(
        dis_ref[...] * (a0_ref[0] + a1_ref[0] + y_ref[...]) + b_ref[...], 0.0)
    p_ref[...] = jnp.dot(h, wa_ref[...], preferred_element_type=_f32)
    q_ref[...] = jnp.dot(h, wb_ref[...], preferred_element_type=_f32)


def _tc_pq(agg, y, dis, b, WA, WB):
    return pl.pallas_call(
        _pq_body,
        grid=(NPAD // _R,),
        in_specs=[
            pl.BlockSpec((1, _R, D), lambda i: (0, i, 0)),
            pl.BlockSpec((1, _R, D), lambda i: (1, i, 0)),
            pl.BlockSpec((_R, D), lambda i: (i, 0)),
            pl.BlockSpec((_R, 1), lambda i: (i, 0)),
            pl.BlockSpec((1, D), lambda i: (0, 0)),
            pl.BlockSpec((D, D), lambda i: (0, 0)),
            pl.BlockSpec((D, D), lambda i: (0, 0)),
        ],
        out_specs=[
            pl.BlockSpec((_R, D), lambda i: (i, 0)),
            pl.BlockSpec((_R, D), lambda i: (i, 0)),
        ],
        out_shape=[
            jax.ShapeDtypeStruct((NPAD, D), _f32),
            jax.ShapeDtypeStruct((NPAD, D), _f32),
        ],
    )(agg, agg, y, dis, b, WA, WB)


_EB = 2000


def _edge_body(pg_ref, qg_ref, ef_ref, wc_ref, b1_ref, w2_ref, b2_ref,
               w3_ref, b3_ref, o_ref):
    z = pg_ref[...] + qg_ref[...] + jnp.dot(
        ef_ref[...], wc_ref[...], preferred_element_type=_f32) + b1_ref[...]
    z = jnp.maximum(z, 0.0)
    z = jnp.maximum(
        jnp.dot(z, w2_ref[...], preferred_element_type=_f32) + b2_ref[...], 0.0)
    o_ref[...] = jnp.dot(z, w3_ref[...], preferred_element_type=_f32) + b3_ref[...]


def _tc_edge(pg, qg, ef, WC, bm1, Wm2, bm2, Wm3, bm3):
    return pl.pallas_call(
        _edge_body,
        grid=(E // _EB,),
        in_specs=[
            pl.BlockSpec((_EB, D), lambda i: (i, 0)),
            pl.BlockSpec((_EB, D), lambda i: (i, 0)),
            pl.BlockSpec((_EB, 16), lambda i: (i, 0)),
            pl.BlockSpec((16, D), lambda i: (0, 0)),
            pl.BlockSpec((1, D), lambda i: (0, 0)),
            pl.BlockSpec((D, 64), lambda i: (0, 0)),
            pl.BlockSpec((1, 64), lambda i: (0, 0)),
            pl.BlockSpec((64, 1), lambda i: (0, 0)),
            pl.BlockSpec((1, 1), lambda i: (0, 0)),
        ],
        out_specs=pl.BlockSpec((_EB, 1), lambda i: (i, 0)),
        out_shape=jax.ShapeDtypeStruct((E, 1), _f32),
    )(pg, qg, ef, WC, bm1, Wm2, bm2, Wm3, bm3)


def kernel(x, edge_index, edge_feat, W1, b1, W2, b2, Wm1, bm1, Wm2, bm2,
           Wm3, bm3):
    src2d = edge_index[0].reshape(EROWS, B)
    dst2d = edge_index[1].reshape(EROWS, B)
    x_pad = jnp.pad(x, ((0, NPAD - N), (0, 0)))
    zeros1 = jnp.zeros((STRIP,), _f32)
    zeros2 = jnp.zeros((STRIP, D), _f32)

    degp = _sc_degree(dst2d, zeros1)
    y1, dis = _tc_prep(x_pad, W1, degp)
    agg1 = _sc_agg(y1, src2d, dst2d, zeros2)
    y2 = _tc_layer(agg1, y1, dis, b1.reshape(1, D), W2)
    agg2 = _sc_agg(y2, src2d, dst2d, zeros2)
    p, q = _tc_pq(agg2, y2, dis, b2.reshape(1, D), Wm1[:D], Wm1[D:2 * D])
    pg, qg = _sc_gather2(p, q, src2d, dst2d)
    return _tc_edge(pg, qg, edge_feat, Wm1[2 * D:], bm1.reshape(1, D),
                    Wm2, bm2.reshape(1, 64), Wm3, bm3.reshape(1, 1))


# speedup vs baseline: 9.4456x; 9.4456x over previous
import functools

import jax
import jax.numpy as jnp
from jax import lax
from jax.experimental import pallas as pl
from jax.experimental.pallas import tpu as pltpu
from jax.experimental.pallas import tpu_sc as plsc

N = 10000
E = 320000
D = 128
NPAD = 10240
STRIP = NPAD // 16
NC, NS = 2, 16
NW = NC * NS
EPT = E // NW
B = 80
RPT = EPT // B
EROWS = E // B

_MESH = plsc.VectorSubcoreMesh(core_axis_name="c", subcore_axis_name="s")
_f32 = jnp.float32


def _wid():
    return lax.axis_index("s") * NC + lax.axis_index("c")


@functools.partial(
    pl.kernel,
    out_type=(jax.ShapeDtypeStruct((NPAD,), _f32),
              jax.ShapeDtypeStruct((NPAD,), _f32)),
    mesh=_MESH,
    scratch_types=[
        pltpu.VMEM((RPT, B), jnp.int32),
        pltpu.VMEM((B,), _f32),
        pltpu.VMEM_SHARED((NPAD,), _f32),
    ],
)
def _sc_degree(dst3d, zeros1, out0, out1, didx, ones_v, acc):
    c = lax.axis_index("c")
    s = lax.axis_index("s")
    pltpu.sync_copy(zeros1, acc.at[pl.ds(s * STRIP, STRIP)])
    pltpu.sync_copy(dst3d.at[_wid()], didx)
    for k in range(B // 16):
        ones_v[pl.ds(k * 16, 16)] = jnp.ones((16,), _f32)
    plsc.subcore_barrier()

    def body(j, carry):
        pltpu.sync_copy(ones_v, acc.at[didx.at[j]], add=True)
        return carry

    lax.fori_loop(0, RPT, body, 0)
    plsc.subcore_barrier()

    @pl.when(c == 0)
    def _():
        pltpu.sync_copy(acc.at[pl.ds(s * STRIP, STRIP)],
                        out0.at[pl.ds(s * STRIP, STRIP)])

    @pl.when(c == 1)
    def _():
        pltpu.sync_copy(acc.at[pl.ds(s * STRIP, STRIP)],
                        out1.at[pl.ds(s * STRIP, STRIP)])


@functools.partial(
    pl.kernel,
    out_type=jax.ShapeDtypeStruct((NC, NPAD, D), _f32),
    mesh=_MESH,
    scratch_types=[
        pltpu.VMEM((RPT, B), jnp.int32),
        pltpu.VMEM((RPT, B), jnp.int32),
        pltpu.VMEM((B, D), _f32),
        pltpu.VMEM_SHARED((NPAD, D), _f32),
        pltpu.SemaphoreType.DMA,
    ],
)
def _sc_agg(y, src3d, dst3d, zeros2, out, sidx, didx, rows, acc, sem):
    c = lax.axis_index("c")
    s = lax.axis_index("s")
    w = _wid()
    pltpu.sync_copy(zeros2, acc.at[pl.ds(s * STRIP, STRIP)])
    pltpu.sync_copy(src3d.at[w], sidx)
    pltpu.sync_copy(dst3d.at[w], didx)
    plsc.subcore_barrier()

    def body(j, carry):
        pltpu.async_copy(y.at[sidx.at[j]], rows, sem).wait()
        pltpu.sync_copy(rows, acc.at[didx.at[j]], add=True)
        return carry

    lax.fori_loop(0, RPT, body, 0)
    plsc.subcore_barrier()
    pltpu.sync_copy(acc.at[pl.ds(s * STRIP, STRIP)],
                    out.at[c, pl.ds(s * STRIP, STRIP)])


@functools.partial(
    pl.kernel,
    out_type=(jax.ShapeDtypeStruct((E, D), _f32),
              jax.ShapeDtypeStruct((E, D), _f32)),
    mesh=_MESH,
    scratch_types=[
        pltpu.VMEM((RPT, B), jnp.int32),
        pltpu.VMEM((RPT, B), jnp.int32),
        pltpu.VMEM((B, D), _f32),
        pltpu.VMEM((B, D), _f32),
        pltpu.SemaphoreType.DMA,
        pltpu.SemaphoreType.DMA,
    ],
)
def _sc_gather2(p, q, src3d, dst3d, pg, qg, sidx, didx, bufp, bufq, semp, semq):
    w = _wid()
    pltpu.sync_copy(src3d.at[w], sidx)
    pltpu.sync_copy(dst3d.at[w], didx)
    base = w * EPT

    def body(j, carry):
        cp = pltpu.async_copy(p.at[sidx.at[j]], bufp, semp)
        cq = pltpu.async_copy(q.at[didx.at[j]], bufq, semq)
        cp.wait()
        cq.wait()
        pltpu.sync_copy(bufp, pg.at[pl.ds(base + j * B, B)])
        pltpu.sync_copy(bufq, qg.at[pl.ds(base + j * B, B)])
        return carry

    lax.fori_loop(0, RPT, body, 0)


_R = 640


def _prep_body(x_ref, w_ref, d0_ref, d1_ref, y_ref, dis_ref):
    dis = lax.rsqrt(d0_ref[...] + d1_ref[...] + 1.0)
    y_ref[...] = jnp.dot(x_ref[...], w_ref[...],
                         preferred_element_type=_f32) * dis
    dis_ref[...] = dis


def _tc_prep(x_pad, W1, d0, d1):
    return pl.pallas_call(
        _prep_body,
        grid=(NPAD // _R,),
        in_specs=[
            pl.BlockSpec((_R, D), lambda i: (i, 0)),
            pl.BlockSpec((D, D), lambda i: (0, 0)),
            pl.BlockSpec((_R, 1), lambda i: (i, 0)),
            pl.BlockSpec((_R, 1), lambda i: (i, 0)),
        ],
        out_specs=[
            pl.BlockSpec((_R, D), lambda i: (i, 0)),
            pl.BlockSpec((_R, 1), lambda i: (i, 0)),
        ],
        out_shape=[
            jax.ShapeDtypeStruct((NPAD, D), _f32),
            jax.ShapeDtypeStruct((NPAD, 1), _f32),
        ],
    )(x_pad, W1, d0, d1)


def _layer_body(a0_ref, a1_ref, y_ref, dis_ref, b_ref, w_ref, o_ref):
    h = jnp.maximum(
        dis_ref[...] * (a0_ref[0] + a1_ref[0] + y_ref[...]) + b_ref[...], 0.0)
    o_ref[...] = jnp.dot(h, w_ref[...], preferred_element_type=_f32) * dis_ref[...]


def _tc_layer(agg, y, dis, b, W):
    return pl.pallas_call(
        _layer_body,
        grid=(NPAD // _R,),
        in_specs=[
            pl.BlockSpec((1, _R, D), lambda i: (0, i, 0)),
            pl.BlockSpec((1, _R, D), lambda i: (1, i, 0)),
            pl.BlockSpec((_R, D), lambda i: (i, 0)),
            pl.BlockSpec((_R, 1), lambda i: (i, 0)),
            pl.BlockSpec((1, D), lambda i: (0, 0)),
            pl.BlockSpec((D, D), lambda i: (0, 0)),
        ],
        out_specs=pl.BlockSpec((_R, D), lambda i: (i, 0)),
        out_shape=jax.ShapeDtypeStruct((NPAD, D), _f32),
    )(agg, agg, y, dis, b, W)


def _pq_body(a0_ref, a1_ref, y_ref, dis_ref, b_ref, wa_ref, wb_ref,
             p_ref, q_ref):
    h = jnp.maximum(
        dis_ref[...] * (a0_ref[0] + a1_ref[0] + y_ref[...]) + b_ref[...], 0.0)
    p_ref[...] = jnp.dot(h, wa_ref[...], preferred_element_type=_f32)
    q_ref[...] = jnp.dot(h, wb_ref[...], preferred_element_type=_f32)


def _tc_pq(agg, y, dis, b, WA, WB):
    return pl.pallas_call(
        _pq_body,
        grid=(NPAD // _R,),
        in_specs=[
            pl.BlockSpec((1, _R, D), lambda i: (0, i, 0)),
            pl.BlockSpec((1, _R, D), lambda i: (1, i, 0)),
            pl.BlockSpec((_R, D), lambda i: (i, 0)),
            pl.BlockSpec((_R, 1), lambda i: (i, 0)),
            pl.BlockSpec((1, D), lambda i: (0, 0)),
            pl.BlockSpec((D, D), lambda i: (0, 0)),
            pl.BlockSpec((D, D), lambda i: (0, 0)),
        ],
        out_specs=[
            pl.BlockSpec((_R, D), lambda i: (i, 0)),
            pl.BlockSpec((_R, D), lambda i: (i, 0)),
        ],
        out_shape=[
            jax.ShapeDtypeStruct((NPAD, D), _f32),
            jax.ShapeDtypeStruct((NPAD, D), _f32),
        ],
    )(agg, agg, y, dis, b, WA, WB)


_EB = 2000


def _edge_body(pg_ref, qg_ref, ef_ref, wc_ref, b1_ref, w2_ref, b2_ref,
               w3_ref, b3_ref, o_ref):
    z = pg_ref[...] + qg_ref[...] + jnp.dot(
        ef_ref[...], wc_ref[...], preferred_element_type=_f32) + b1_ref[...]
    z = jnp.maximum(z, 0.0)
    z = jnp.maximum(
        jnp.dot(z, w2_ref[...], preferred_element_type=_f32) + b2_ref[...], 0.0)
    o_ref[...] = jnp.dot(z, w3_ref[...], preferred_element_type=_f32) + b3_ref[...]


def _tc_edge(pg, qg, ef, WC, bm1, Wm2, bm2, Wm3, bm3):
    return pl.pallas_call(
        _edge_body,
        grid=(E // _EB,),
        in_specs=[
            pl.BlockSpec((_EB, D), lambda i: (i, 0)),
            pl.BlockSpec((_EB, D), lambda i: (i, 0)),
            pl.BlockSpec((_EB, 16), lambda i: (i, 0)),
            pl.BlockSpec((16, D), lambda i: (0, 0)),
            pl.BlockSpec((1, D), lambda i: (0, 0)),
            pl.BlockSpec((D, 64), lambda i: (0, 0)),
            pl.BlockSpec((1, 64), lambda i: (0, 0)),
            pl.BlockSpec((64, 1), lambda i: (0, 0)),
            pl.BlockSpec((1, 1), lambda i: (0, 0)),
        ],
        out_specs=pl.BlockSpec((_EB, 1), lambda i: (i, 0)),
        out_shape=jax.ShapeDtypeStruct((E, 1), _f32),
    )(pg, qg, ef, WC, bm1, Wm2, bm2, Wm3, bm3)


def kernel(x, edge_index, edge_feat, W1, b1, W2, b2, Wm1, bm1, Wm2, bm2,
           Wm3, bm3):
    src3d = edge_index[0].reshape(NW, RPT, B)
    dst3d = edge_index[1].reshape(NW, RPT, B)
    x_pad = jnp.pad(x, ((0, NPAD - N), (0, 0)))
    zeros1 = jnp.zeros((STRIP,), _f32)
    zeros2 = jnp.zeros((STRIP, D), _f32)

    d0, d1 = _sc_degree(dst3d, zeros1)
    y1, dis = _tc_prep(x_pad, W1, d0.reshape(NPAD, 1), d1.reshape(NPAD, 1))
    agg1 = _sc_agg(y1, src3d, dst3d, zeros2)
    y2 = _tc_layer(agg1, y1, dis, b1.reshape(1, D), W2)
    agg2 = _sc_agg(y2, src3d, dst3d, zeros2)
    p, q = _tc_pq(agg2, y2, dis, b2.reshape(1, D), Wm1[:D], Wm1[D:2 * D])
    pg, qg = _sc_gather2(p, q, src3d, dst3d)
    return _tc_edge(pg, qg, edge_feat, Wm1[2 * D:], bm1.reshape(1, D),
                    Wm2, bm2.reshape(1, 64), Wm3, bm3.reshape(1, 1))
